# baseline (speedup 1.0000x reference)
import jax
import jax.numpy as jnp
from jax.experimental import pallas as pl
from jax.experimental.pallas import tpu as pltpu

BATCH = 4
SEQ_LEN = 8192
EMBED_DIM = 768
POOL_SIZE = 64
LENGTH = 5
TOP_K = 8
TASK_PROMPT_SIZE = 8

SEQ_BLK = 512
N_SEQ_BLK = SEQ_LEN // SEQ_BLK
NBLK = BATCH * N_SEQ_BLK
NBUF = 12
LOOKAHEAD = 6
HEAD_ROWS = (TASK_PROMPT_SIZE + TOP_K) * LENGTH
OUT_ROWS = HEAD_ROWS + SEQ_LEN
_HI = jax.lax.Precision.HIGHEST


def _routing(sim0, pr, ar, idx_ref, bmp_ref, rsum_ref):
    iota64 = jax.lax.broadcasted_iota(
        jnp.int32, (BATCH, POOL_SIZE), 1).astype(jnp.float32)

    sim = sim0
    cols = []
    for _ in range(TOP_K):
        m = jnp.max(sim, axis=1, keepdims=True)
        cand = jnp.where(sim == m, iota64, 1e9)
        i0 = jnp.min(cand, axis=1, keepdims=True)
        cols.append(i0)
        sim = jnp.where(iota64 == i0, -jnp.inf, sim)
    idxf = jnp.concatenate(cols, axis=1)
    idx_ref[:, :] = idxf.astype(jnp.int32)

    jj = jax.lax.broadcasted_iota(jnp.int32, (TOP_K, TOP_K * LENGTH), 1)
    pp = jax.lax.broadcasted_iota(jnp.int32, (TOP_K, TOP_K * LENGTH), 0)
    expand = (jj // LENGTH == pp).astype(jnp.float32)
    lvec = (jj[0:1, :] % LENGTH).astype(jnp.float32)
    colsel = jax.lax.dot_general(
        idxf, expand, (((1,), (0,)), ((), ())),
        preferred_element_type=jnp.float32, precision=_HI,
    ) * LENGTH + lvec

    nsel = TOP_K * LENGTH
    eye_r = jax.lax.broadcasted_iota(jnp.int32, (nsel, nsel), 0)
    eye_c = jax.lax.broadcasted_iota(jnp.int32, (nsel, nsel), 1)
    eye = (eye_r == eye_c).astype(jnp.float32)
    riota = jax.lax.broadcasted_iota(
        jnp.int32, (nsel, POOL_SIZE * LENGTH), 1).astype(jnp.float32)

    for b in range(BATCH):
        cb = colsel[b:b + 1, :]
        cbt = jax.lax.dot_general(
            eye, cb, (((1,), (1,)), ((), ())),
            preferred_element_type=jnp.float32, precision=_HI,
        )
        onehot = (jnp.broadcast_to(cbt, (nsel, POOL_SIZE * LENGTH)) ==
                  riota).astype(jnp.float32)
        part = jax.lax.dot_general(
            onehot, pr, (((1,), (0,)), ((), ())),
            preferred_element_type=jnp.float32, precision=_HI,
        )
        bmp_ref[b, 0:TASK_PROMPT_SIZE * LENGTH, :] = ar
        bmp_ref[b, TASK_PROMPT_SIZE * LENGTH:HEAD_ROWS, :] = part

    cacc = jnp.zeros((BATCH, POOL_SIZE), jnp.float32)
    for k in range(TOP_K):
        cacc = cacc + (idxf[:, k:k + 1] == iota64).astype(jnp.float32)
    counts = jnp.sum(cacc, axis=0, keepdims=True)
    colsum = jnp.sum(sim0, axis=0, keepdims=True)
    rsum_ref[0, 0] = jnp.sum(counts * colsum) * (1.0 / BATCH)


def _fused(x_hbm, pk_ref, pr_ref, ar_ref,
           out_hbm, xnorm_ref, idx_ref, bmp_ref, rsum_ref,
           buf, acc_ref, rsem, wsem, hsem):
    b = pl.program_id(0)
    s = pl.program_id(1)
    i = b * N_SEQ_BLK + s

    def read_cp(j):
        bb = j // N_SEQ_BLK
        ss = j - bb * N_SEQ_BLK
        return pltpu.make_async_copy(
            x_hbm.at[pl.ds(bb, 1), pl.ds(ss * SEQ_BLK, SEQ_BLK), :],
            buf.at[pl.ds(j % NBUF, 1)],
            rsem.at[j % NBUF],
        )

    def write_cp(j):
        bb = j // N_SEQ_BLK
        ss = j - bb * N_SEQ_BLK
        return pltpu.make_async_copy(
            buf.at[pl.ds(j % NBUF, 1)],
            out_hbm.at[pl.ds(bb, 1),
                       pl.ds(HEAD_ROWS + ss * SEQ_BLK, SEQ_BLK), :],
            wsem.at[j % NBUF],
        )

    @pl.when(i == 0)
    def _():
        for j in range(LOOKAHEAD):
            read_cp(j).start()

    read_cp(i).wait()
    write_cp(i).start()
    psum = jnp.sum(buf[i % NBUF], axis=0, keepdims=True)

    @pl.when(s == 0)
    def _():
        acc_ref[0:1, :] = psum

    @pl.when(s > 0)
    def _():
        acc_ref[0:1, :] = acc_ref[0:1, :] + psum

    @pl.when(i >= NBUF - LOOKAHEAD)
    def _():
        write_cp(i - (NBUF - LOOKAHEAD)).wait()

    @pl.when(i + LOOKAHEAD < NBLK)
    def _():
        read_cp(i + LOOKAHEAD).start()

    @pl.when(s == N_SEQ_BLK - 1)
    def _():
        mean = acc_ref[0:1, :] * (1.0 / SEQ_LEN)
        ss = jnp.sum(mean * mean, axis=1, keepdims=True)
        xn = mean * jax.lax.rsqrt(jnp.maximum(ss, 1e-12))
        xnorm_ref[pl.ds(b, 1), :] = xn

    @pl.when((b == BATCH - 1) & (s == N_SEQ_BLK - 1))
    def _():
        pk = pk_ref[:, :]
        pss = jnp.sum(pk * pk, axis=1, keepdims=True)
        pn = pk * jax.lax.rsqrt(jnp.maximum(pss, 1e-12))
        sim0 = jax.lax.dot_general(
            xnorm_ref[:, :], pn, (((1,), (1,)), ((), ())),
            preferred_element_type=jnp.float32,
        )
        _routing(sim0, pr_ref[:, :], ar_ref[:, :], idx_ref, bmp_ref, rsum_ref)

        head = pltpu.make_async_copy(
            bmp_ref, out_hbm.at[:, pl.ds(0, HEAD_ROWS), :], hsem)
        head.start()
        head.wait()

        for k in range(NBUF - LOOKAHEAD):
            write_cp(NBLK - 1 - k).wait()


def kernel(x_embed, prompt, prompt_key, assist_prompt, test=1, threshold=-2):
    prompt_r = prompt.reshape(POOL_SIZE * LENGTH, EMBED_DIM)
    assist_r = assist_prompt.reshape(TASK_PROMPT_SIZE * LENGTH, EMBED_DIM)

    prompted, xnorm, idx, bmp, rsum = pl.pallas_call(
        _fused,
        grid=(BATCH, N_SEQ_BLK),
        in_specs=[
            pl.BlockSpec(memory_space=pl.ANY),
            pl.BlockSpec((POOL_SIZE, EMBED_DIM), lambda b, s: (0, 0)),
            pl.BlockSpec((POOL_SIZE * LENGTH, EMBED_DIM), lambda b, s: (0, 0)),
            pl.BlockSpec((TASK_PROMPT_SIZE * LENGTH, EMBED_DIM),
                         lambda b, s: (0, 0)),
        ],
        out_specs=[
            pl.BlockSpec(memory_space=pl.ANY),
            pl.BlockSpec((BATCH, EMBED_DIM), lambda b, s: (0, 0)),
            pl.BlockSpec((BATCH, TOP_K), lambda b, s: (0, 0)),
            pl.BlockSpec((BATCH, HEAD_ROWS, EMBED_DIM), lambda b, s: (0, 0, 0)),
            pl.BlockSpec(memory_space=pltpu.SMEM),
        ],
        out_shape=[
            jax.ShapeDtypeStruct((BATCH, OUT_ROWS, EMBED_DIM), jnp.float32),
            jax.ShapeDtypeStruct((BATCH, EMBED_DIM), jnp.float32),
            jax.ShapeDtypeStruct((BATCH, TOP_K), jnp.int32),
            jax.ShapeDtypeStruct((BATCH, HEAD_ROWS, EMBED_DIM), jnp.float32),
            jax.ShapeDtypeStruct((1, 1), jnp.float32),
        ],
        scratch_shapes=[
            pltpu.VMEM((NBUF, SEQ_BLK, EMBED_DIM), jnp.float32),
            pltpu.VMEM((8, EMBED_DIM), jnp.float32),
            pltpu.SemaphoreType.DMA((NBUF,)),
            pltpu.SemaphoreType.DMA((NBUF,)),
            pltpu.SemaphoreType.DMA,
        ],
    )(x_embed, prompt_key, prompt_r, assist_r)

    return prompted, rsum.reshape(()), bmp, xnorm, idx


# speedup vs baseline: 1.0333x; 1.0333x over previous
import jax
import jax.numpy as jnp
from jax.experimental import pallas as pl
from jax.experimental.pallas import tpu as pltpu

BATCH = 4
SEQ_LEN = 8192
EMBED_DIM = 768
POOL_SIZE = 64
LENGTH = 5
TOP_K = 8
TASK_PROMPT_SIZE = 8

SEQ_BLK = 512
N_SEQ_BLK = SEQ_LEN // SEQ_BLK
NBLK = BATCH * N_SEQ_BLK
NBUF = 12
LOOKAHEAD = 6
HEAD_ROWS = (TASK_PROMPT_SIZE + TOP_K) * LENGTH
OUT_ROWS = HEAD_ROWS + SEQ_LEN
_HI = jax.lax.Precision.HIGHEST


def _batch_routing(b, sim_b, pr, ar, idx_ref, bmp_ref):
    iota64 = jax.lax.broadcasted_iota(
        jnp.int32, (1, POOL_SIZE), 1).astype(jnp.float32)

    sim = sim_b
    cols = []
    for _ in range(TOP_K):
        m = jnp.max(sim, axis=1, keepdims=True)
        cand = jnp.where(sim == m, iota64, 1e9)
        i0 = jnp.min(cand, axis=1, keepdims=True)
        cols.append(i0)
        sim = jnp.where(iota64 == i0, -jnp.inf, sim)
    idxf = jnp.concatenate(cols, axis=1)
    idx_ref[pl.ds(b, 1), :] = idxf.astype(jnp.int32)

    jj = jax.lax.broadcasted_iota(jnp.int32, (TOP_K, TOP_K * LENGTH), 1)
    pp = jax.lax.broadcasted_iota(jnp.int32, (TOP_K, TOP_K * LENGTH), 0)
    expand = (jj // LENGTH == pp).astype(jnp.float32)
    lvec = (jj[0:1, :] % LENGTH).astype(jnp.float32)
    colsel = jax.lax.dot_general(
        idxf, expand, (((1,), (0,)), ((), ())),
        preferred_element_type=jnp.float32, precision=_HI,
    ) * LENGTH + lvec

    nsel = TOP_K * LENGTH
    eye_r = jax.lax.broadcasted_iota(jnp.int32, (nsel, nsel), 0)
    eye_c = jax.lax.broadcasted_iota(jnp.int32, (nsel, nsel), 1)
    eye = (eye_r == eye_c).astype(jnp.float32)
    riota = jax.lax.broadcasted_iota(
        jnp.int32, (nsel, POOL_SIZE * LENGTH), 1).astype(jnp.float32)

    cbt = jax.lax.dot_general(
        eye, colsel, (((1,), (1,)), ((), ())),
        preferred_element_type=jnp.float32, precision=_HI,
    )
    onehot = (jnp.broadcast_to(cbt, (nsel, POOL_SIZE * LENGTH)) ==
              riota).astype(jnp.float32)
    part = jax.lax.dot_general(
        onehot, pr, (((1,), (0,)), ((), ())),
        preferred_element_type=jnp.float32, precision=_HI,
    )
    bmp_ref[pl.ds(b, 1), 0:TASK_PROMPT_SIZE * LENGTH, :] = (
        ar.reshape(1, TASK_PROMPT_SIZE * LENGTH, EMBED_DIM))
    bmp_ref[pl.ds(b, 1), TASK_PROMPT_SIZE * LENGTH:HEAD_ROWS, :] = (
        part.reshape(1, TOP_K * LENGTH, EMBED_DIM))


def _fused(x_hbm, pk_ref, pr_ref, ar_ref,
           out_hbm, xnorm_ref, idx_ref, bmp_ref, rsum_ref,
           buf, acc_ref, sim_scr, rsem, wsem, hsem):
    b = pl.program_id(0)
    s = pl.program_id(1)
    i = b * N_SEQ_BLK + s

    def read_cp(j):
        bb = j // N_SEQ_BLK
        ss = j - bb * N_SEQ_BLK
        return pltpu.make_async_copy(
            x_hbm.at[pl.ds(bb, 1), pl.ds(ss * SEQ_BLK, SEQ_BLK), :],
            buf.at[pl.ds(j % NBUF, 1)],
            rsem.at[j % NBUF],
        )

    def write_cp(j):
        bb = j // N_SEQ_BLK
        ss = j - bb * N_SEQ_BLK
        return pltpu.make_async_copy(
            buf.at[pl.ds(j % NBUF, 1)],
            out_hbm.at[pl.ds(bb, 1),
                       pl.ds(HEAD_ROWS + ss * SEQ_BLK, SEQ_BLK), :],
            wsem.at[j % NBUF],
        )

    def head_cp(bb):
        return pltpu.make_async_copy(
            bmp_ref.at[pl.ds(bb, 1)],
            out_hbm.at[pl.ds(bb, 1), pl.ds(0, HEAD_ROWS), :],
            hsem,
        )

    @pl.when(i == 0)
    def _():
        for j in range(LOOKAHEAD):
            read_cp(j).start()

    read_cp(i).wait()
    write_cp(i).start()
    psum = jnp.sum(buf[i % NBUF], axis=0, keepdims=True)

    @pl.when(s == 0)
    def _():
        acc_ref[0:1, :] = psum

    @pl.when(s > 0)
    def _():
        acc_ref[0:1, :] = acc_ref[0:1, :] + psum

    @pl.when(i >= NBUF - LOOKAHEAD)
    def _():
        write_cp(i - (NBUF - LOOKAHEAD)).wait()

    @pl.when(i + LOOKAHEAD < NBLK)
    def _():
        read_cp(i + LOOKAHEAD).start()

    @pl.when(s == N_SEQ_BLK - 1)
    def _():
        mean = acc_ref[0:1, :] * (1.0 / SEQ_LEN)
        ss = jnp.sum(mean * mean, axis=1, keepdims=True)
        xn = mean * jax.lax.rsqrt(jnp.maximum(ss, 1e-12))
        xnorm_ref[pl.ds(b, 1), :] = xn

        pk = pk_ref[:, :]
        pss = jnp.sum(pk * pk, axis=1, keepdims=True)
        pn = pk * jax.lax.rsqrt(jnp.maximum(pss, 1e-12))
        sim_b = jax.lax.dot_general(
            xn, pn, (((1,), (1,)), ((), ())),
            preferred_element_type=jnp.float32,
        )
        sim_scr[pl.ds(b, 1), :] = sim_b
        _batch_routing(b, sim_b, pr_ref[:, :], ar_ref[:, :], idx_ref, bmp_ref)
        head_cp(b).start()

    @pl.when((b == BATCH - 1) & (s == N_SEQ_BLK - 1))
    def _():
        iota64 = jax.lax.broadcasted_iota(
            jnp.int32, (BATCH, POOL_SIZE), 1).astype(jnp.float32)
        idxf = idx_ref[:, :].astype(jnp.float32)
        cacc = jnp.zeros((BATCH, POOL_SIZE), jnp.float32)
        for k in range(TOP_K):
            cacc = cacc + (idxf[:, k:k + 1] == iota64).astype(jnp.float32)
        counts = jnp.sum(cacc, axis=0, keepdims=True)
        colsum = jnp.sum(sim_scr[:, :], axis=0, keepdims=True)
        rsum_ref[0, 0] = jnp.sum(counts * colsum) * (1.0 / BATCH)

        for bb in range(BATCH):
            head_cp(bb).wait()
        for k in range(NBUF - LOOKAHEAD):
            write_cp(NBLK - 1 - k).wait()


def kernel(x_embed, prompt, prompt_key, assist_prompt, test=1, threshold=-2):
    prompt_r = prompt.reshape(POOL_SIZE * LENGTH, EMBED_DIM)
    assist_r = assist_prompt.reshape(TASK_PROMPT_SIZE * LENGTH, EMBED_DIM)

    prompted, xnorm, idx, bmp, rsum = pl.pallas_call(
        _fused,
        grid=(BATCH, N_SEQ_BLK),
        in_specs=[
            pl.BlockSpec(memory_space=pl.ANY),
            pl.BlockSpec((POOL_SIZE, EMBED_DIM), lambda b, s: (0, 0)),
            pl.BlockSpec((POOL_SIZE * LENGTH, EMBED_DIM), lambda b, s: (0, 0)),
            pl.BlockSpec((TASK_PROMPT_SIZE * LENGTH, EMBED_DIM),
                         lambda b, s: (0, 0)),
        ],
        out_specs=[
            pl.BlockSpec(memory_space=pl.ANY),
            pl.BlockSpec((BATCH, EMBED_DIM), lambda b, s: (0, 0)),
            pl.BlockSpec((BATCH, TOP_K), lambda b, s: (0, 0)),
            pl.BlockSpec((BATCH, HEAD_ROWS, EMBED_DIM), lambda b, s: (0, 0, 0)),
            pl.BlockSpec(memory_space=pltpu.SMEM),
        ],
        out_shape=[
            jax.ShapeDtypeStruct((BATCH, OUT_ROWS, EMBED_DIM), jnp.float32),
            jax.ShapeDtypeStruct((BATCH, EMBED_DIM), jnp.float32),
            jax.ShapeDtypeStruct((BATCH, TOP_K), jnp.int32),
            jax.ShapeDtypeStruct((BATCH, HEAD_ROWS, EMBED_DIM), jnp.float32),
            jax.ShapeDtypeStruct((1, 1), jnp.float32),
        ],
        scratch_shapes=[
            pltpu.VMEM((NBUF, SEQ_BLK, EMBED_DIM), jnp.float32),
            pltpu.VMEM((8, EMBED_DIM), jnp.float32),
            pltpu.VMEM((BATCH, POOL_SIZE), jnp.float32),
            pltpu.SemaphoreType.DMA((NBUF,)),
            pltpu.SemaphoreType.DMA((NBUF,)),
            pltpu.SemaphoreType.DMA,
        ],
    )(x_embed, prompt_key, prompt_r, assist_r)

    return prompted, rsum.reshape(()), bmp, xnorm, idx
